# baseline (speedup 1.0000x reference)
import functools
import math

import jax
import jax.numpy as jnp
from jax import lax
from jax.experimental import pallas as pl
from jax.experimental.pallas import tpu as pltpu
from jax.experimental.pallas import tpu_sc as plsc

_BATCH = 4096
_LSEQ = 50
_D = 32
_NC, _NS = 2, 16
_NW = _NC * _NS
_BPW = _BATCH // _NW
_UFULL = 7812
_PAIRS = _UFULL // (2 * _NW)
_SQRT_S = math.sqrt(32.0)
_L = 16
_SROWS = 250016

_MESH = plsc.VectorSubcoreMesh(core_axis_name="c", subcore_axis_name="s")
_PARAMS = pltpu.CompilerParams(needs_layout_passes=False)


def _worker_id():
    return lax.axis_index("s") * _NC + lax.axis_index("c")


@functools.partial(
    pl.kernel,
    mesh=_MESH,
    compiler_params=_PARAMS,
    out_type=jax.ShapeDtypeStruct((_SROWS, 128), jnp.float32),
    scratch_types=[
        pltpu.VMEM((2, _D, 128), jnp.float32),
        pltpu.VMEM((2, _D, 128), jnp.float32),
        pltpu.SemaphoreType.DMA,
        pltpu.SemaphoreType.DMA,
        pltpu.SemaphoreType.DMA,
        pltpu.SemaphoreType.DMA,
    ],
)
def _transpose_sc(tT_hbm, tail_hbm, scr_hbm, bin_v, bout_v, sf0, sf1, sw0, sw1):
    wid = _worker_id()
    lanes = lax.iota(jnp.int32, _L)

    def transpose_block(ib, ob, ncg):
        @plsc.parallel_loop(0, ncg, unroll=1)
        def _(cg):
            c16 = cg * _L + lanes
            drow = lax.shift_right_logical(c16, 2)
            dcol = lax.shift_left(jnp.bitwise_and(c16, 3), 5)
            for d in range(_D):
                vals = plsc.load_gather(
                    ib, [jnp.broadcast_to(jnp.int32(d), (_L,)), c16]
                )
                plsc.store_scatter(ob, [drow, dcol + d], vals * _SQRT_S)

    def pair_body(k2, carry):
        u_a = wid + (2 * k2) * _NW
        u_b = u_a + _NW
        fa = pltpu.async_copy(
            tT_hbm.at[:, pl.ds(u_a * 128, 128)], bin_v.at[0], sf0
        )
        fb = pltpu.async_copy(
            tT_hbm.at[:, pl.ds(u_b * 128, 128)], bin_v.at[1], sf1
        )
        fa.wait()
        transpose_block(bin_v.at[0], bout_v.at[0], 8)
        wa = pltpu.async_copy(
            bout_v.at[0], scr_hbm.at[pl.ds(u_a * _D, _D)], sw0
        )
        fb.wait()
        transpose_block(bin_v.at[1], bout_v.at[1], 8)
        wb = pltpu.async_copy(
            bout_v.at[1], scr_hbm.at[pl.ds(u_b * _D, _D)], sw1
        )
        wa.wait()
        wb.wait()
        return carry

    lax.fori_loop(0, _PAIRS, pair_body, 0)

    @pl.when(wid < 4)
    def _():
        u = _UFULL - 4 + wid
        pltpu.async_copy(
            tT_hbm.at[:, pl.ds(u * 128, 128)], bin_v.at[0], sf0
        ).wait()
        transpose_block(bin_v.at[0], bout_v.at[0], 8)
        pltpu.async_copy(
            bout_v.at[0], scr_hbm.at[pl.ds(u * _D, _D)], sw0
        ).wait()

    @pl.when(wid == 4)
    def _():
        pltpu.async_copy(tail_hbm, bin_v.at[0], sf0).wait()
        transpose_block(bin_v.at[0], bout_v.at[0], 8)
        pltpu.async_copy(
            bout_v.at[0], scr_hbm.at[pl.ds(_UFULL * _D, _D)], sw0
        ).wait()


@functools.partial(
    pl.kernel,
    mesh=_MESH,
    compiler_params=_PARAMS,
    out_type=jax.ShapeDtypeStruct((_LSEQ, 4, _NW, 8, _BPW), jnp.float32),
    scratch_types=[
        pltpu.VMEM((_LSEQ, _BPW), jnp.int32),
        pltpu.VMEM((_LSEQ * _BPW,), jnp.int32),
        pltpu.VMEM((_LSEQ * _BPW,), jnp.int32),
        pltpu.VMEM((2, _BPW, 128), jnp.float32),
        pltpu.VMEM((2, 1, 4, 1, 8, _BPW), jnp.float32),
        pltpu.SemaphoreType.DMA,
        pltpu.SemaphoreType.DMA,
        pltpu.SemaphoreType.DMA,
        pltpu.SemaphoreType.DMA,
    ],
)
def _gather_sc(
    xT_hbm, scr_hbm, out_hbm,
    xsl_v, m_v, sub_v, wide_v, outc_v, sg0, sg1, ss0, ss1,
):
    wid = _worker_id()
    lanes = lax.iota(jnp.int32, _L)

    pltpu.sync_copy(xT_hbm.at[:, pl.ds(wid * _BPW, _BPW)], xsl_v)

    @plsc.parallel_loop(0, _LSEQ * _BPW // _L, unroll=1)
    def _(g):
        r = lax.shift_right_logical(g, 3)
        col = jnp.bitwise_and(g, 7) * _L
        v = xsl_v[r, pl.ds(col, _L)]
        m_v[pl.ds(g * _L, _L)] = lax.shift_right_logical(v, 2)
        sub_v[pl.ds(g * _L, _L)] = lax.shift_left(jnp.bitwise_and(v, 3), 5)

    def start_gather(l, buf, sem):
        return pltpu.async_copy(
            scr_hbm.at[m_v.at[pl.ds(l * _BPW, _BPW)]], wide_v.at[buf], sem
        )

    def wait_gather(l, buf, sem):
        pltpu.make_async_copy(
            scr_hbm.at[m_v.at[pl.ds(l * _BPW, _BPW)]], wide_v.at[buf], sem
        ).wait()

    def rearrange(l, buf):
        wbuf = wide_v.at[buf]
        obuf = outc_v.at[buf]

        @plsc.parallel_loop(0, _BPW // _L, unroll=1)
        def _(cg):
            rows16 = cg * _L + lanes
            sub16 = sub_v[pl.ds(l * _BPW + cg * _L, _L)]
            cslice = pl.ds(cg * _L, _L)
            for d in range(_D):
                vals = plsc.load_gather(wbuf, [rows16, sub16 + d])
                obuf[0, d >> 3, 0, d & 7, cslice] = vals

    def store_out(l, buf, sem):
        return pltpu.async_copy(
            outc_v.at[buf],
            out_hbm.at[pl.ds(l, 1), pl.ds(0, 4), pl.ds(wid, 1)],
            sem,
        )

    def wait_store(l, buf, sem):
        pltpu.make_async_copy(
            outc_v.at[buf],
            out_hbm.at[pl.ds(l, 1), pl.ds(0, 4), pl.ds(wid, 1)],
            sem,
        ).wait()

    start_gather(0, 0, sg0)

    def p2_body(k2, carry):
        l0 = 2 * k2
        l1 = l0 + 1
        start_gather(l1, 1, sg1)
        wait_gather(l0, 0, sg0)

        @pl.when(k2 > 0)
        def _():
            wait_store(l0 - 2, 0, ss0)

        rearrange(l0, 0)
        store_out(l0, 0, ss0)

        @pl.when(l0 + 2 < _LSEQ)
        def _():
            start_gather(l0 + 2, 0, sg0)

        wait_gather(l1, 1, sg1)

        @pl.when(k2 > 0)
        def _():
            wait_store(l1 - 2, 1, ss1)

        rearrange(l1, 1)
        store_out(l1, 1, ss1)
        return carry

    lax.fori_loop(0, _LSEQ // 2, p2_body, 0)

    wait_store(_LSEQ - 2, 0, ss0)
    wait_store(_LSEQ - 1, 1, ss1)


def kernel(x, table):
    xT = x.astype(jnp.int32).T
    tT = table.T
    tail = jnp.pad(tT[:, _UFULL * 128 :], ((0, 0), (0, 64)))
    scratch = _transpose_sc(tT, tail)
    out5 = _gather_sc(xT, scratch)
    out = out5.transpose(2, 4, 0, 1, 3).reshape(_BATCH, _LSEQ, _D)
    return out


# speedup vs baseline: 1.1148x; 1.1148x over previous
import functools
import math

import jax
import jax.numpy as jnp
from jax import lax
from jax.experimental import pallas as pl
from jax.experimental.pallas import tpu as pltpu
from jax.experimental.pallas import tpu_sc as plsc

_BATCH = 4096
_LSEQ = 50
_D = 32
_NC, _NS = 2, 16
_NW = _NC * _NS
_BPW = _BATCH // _NW
_UFULL = 7812
_PAIRS = _UFULL // (2 * _NW)
_SQRT_S = math.sqrt(32.0)
_L = 16
_SROWS = 250016

_MESH = plsc.VectorSubcoreMesh(core_axis_name="c", subcore_axis_name="s")
_PARAMS = pltpu.CompilerParams(needs_layout_passes=False)


def _worker_id():
    return lax.axis_index("s") * _NC + lax.axis_index("c")


@functools.partial(
    pl.kernel,
    mesh=_MESH,
    compiler_params=_PARAMS,
    out_type=jax.ShapeDtypeStruct((_SROWS, 128), jnp.float32),
    scratch_types=[
        pltpu.VMEM((2, _D, 128), jnp.float32),
        pltpu.VMEM((2, _D, 128), jnp.float32),
        pltpu.SemaphoreType.DMA,
        pltpu.SemaphoreType.DMA,
        pltpu.SemaphoreType.DMA,
        pltpu.SemaphoreType.DMA,
    ],
)
def _transpose_sc(tT_hbm, tail_hbm, scr_hbm, bin_v, bout_v, sf0, sf1, sw0, sw1):
    wid = _worker_id()
    lanes = lax.iota(jnp.int32, _L)

    def transpose_block(ib, ob, ncg):
        @plsc.parallel_loop(0, ncg, unroll=1)
        def _(cg):
            c16 = cg * _L + lanes
            drow = lax.shift_right_logical(c16, 2)
            dcol = lax.shift_left(jnp.bitwise_and(c16, 3), 5)
            for d0 in range(0, _D, 8):
                vals = [
                    plsc.load_gather(
                        ib, [jnp.broadcast_to(jnp.int32(d), (_L,)), c16]
                    )
                    for d in range(d0, d0 + 8)
                ]
                for i, d in enumerate(range(d0, d0 + 8)):
                    plsc.store_scatter(ob, [drow, dcol + d], vals[i] * _SQRT_S)

    def pair_body(k2, carry):
        u_a = wid + (2 * k2) * _NW
        u_b = u_a + _NW
        fa = pltpu.async_copy(
            tT_hbm.at[:, pl.ds(u_a * 128, 128)], bin_v.at[0], sf0
        )
        fb = pltpu.async_copy(
            tT_hbm.at[:, pl.ds(u_b * 128, 128)], bin_v.at[1], sf1
        )
        fa.wait()
        transpose_block(bin_v.at[0], bout_v.at[0], 8)
        wa = pltpu.async_copy(
            bout_v.at[0], scr_hbm.at[pl.ds(u_a * _D, _D)], sw0
        )
        fb.wait()
        transpose_block(bin_v.at[1], bout_v.at[1], 8)
        wb = pltpu.async_copy(
            bout_v.at[1], scr_hbm.at[pl.ds(u_b * _D, _D)], sw1
        )
        wa.wait()
        wb.wait()
        return carry

    lax.fori_loop(0, _PAIRS, pair_body, 0)

    @pl.when(wid < 4)
    def _():
        u = _UFULL - 4 + wid
        pltpu.async_copy(
            tT_hbm.at[:, pl.ds(u * 128, 128)], bin_v.at[0], sf0
        ).wait()
        transpose_block(bin_v.at[0], bout_v.at[0], 8)
        pltpu.async_copy(
            bout_v.at[0], scr_hbm.at[pl.ds(u * _D, _D)], sw0
        ).wait()

    @pl.when(wid == 4)
    def _():
        pltpu.async_copy(tail_hbm, bin_v.at[0], sf0).wait()
        transpose_block(bin_v.at[0], bout_v.at[0], 8)
        pltpu.async_copy(
            bout_v.at[0], scr_hbm.at[pl.ds(_UFULL * _D, _D)], sw0
        ).wait()


@functools.partial(
    pl.kernel,
    mesh=_MESH,
    compiler_params=_PARAMS,
    out_type=jax.ShapeDtypeStruct((_LSEQ, 4, _NW, 8, _BPW), jnp.float32),
    scratch_types=[
        pltpu.VMEM((_LSEQ, _BPW), jnp.int32),
        pltpu.VMEM((_LSEQ * _BPW,), jnp.int32),
        pltpu.VMEM((_LSEQ * _BPW,), jnp.int32),
        pltpu.VMEM((2, _BPW, 128), jnp.float32),
        pltpu.VMEM((2, 1, 4, 1, 8, _BPW), jnp.float32),
        pltpu.SemaphoreType.DMA,
        pltpu.SemaphoreType.DMA,
        pltpu.SemaphoreType.DMA,
        pltpu.SemaphoreType.DMA,
    ],
)
def _gather_sc(
    xT_hbm, scr_hbm, out_hbm,
    xsl_v, m_v, sub_v, wide_v, outc_v, sg0, sg1, ss0, ss1,
):
    wid = _worker_id()
    lanes = lax.iota(jnp.int32, _L)

    pltpu.sync_copy(xT_hbm.at[:, pl.ds(wid * _BPW, _BPW)], xsl_v)

    @plsc.parallel_loop(0, _LSEQ * _BPW // _L, unroll=1)
    def _(g):
        r = lax.shift_right_logical(g, 3)
        col = jnp.bitwise_and(g, 7) * _L
        v = xsl_v[r, pl.ds(col, _L)]
        m_v[pl.ds(g * _L, _L)] = lax.shift_right_logical(v, 2)
        sub_v[pl.ds(g * _L, _L)] = lax.shift_left(jnp.bitwise_and(v, 3), 5)

    def start_gather(l, buf, sem):
        return pltpu.async_copy(
            scr_hbm.at[m_v.at[pl.ds(l * _BPW, _BPW)]], wide_v.at[buf], sem
        )

    def wait_gather(l, buf, sem):
        pltpu.make_async_copy(
            scr_hbm.at[m_v.at[pl.ds(l * _BPW, _BPW)]], wide_v.at[buf], sem
        ).wait()

    def rearrange(l, buf):
        wbuf = wide_v.at[buf]
        obuf = outc_v.at[buf]

        @plsc.parallel_loop(0, _BPW // _L, unroll=1)
        def _(cg):
            rows16 = cg * _L + lanes
            sub16 = sub_v[pl.ds(l * _BPW + cg * _L, _L)]
            cslice = pl.ds(cg * _L, _L)
            for d0 in range(0, _D, 8):
                vals = [
                    plsc.load_gather(wbuf, [rows16, sub16 + d])
                    for d in range(d0, d0 + 8)
                ]
                for i, d in enumerate(range(d0, d0 + 8)):
                    obuf[0, d >> 3, 0, d & 7, cslice] = vals[i]

    def store_out(l, buf, sem):
        return pltpu.async_copy(
            outc_v.at[buf],
            out_hbm.at[pl.ds(l, 1), pl.ds(0, 4), pl.ds(wid, 1)],
            sem,
        )

    def wait_store(l, buf, sem):
        pltpu.make_async_copy(
            outc_v.at[buf],
            out_hbm.at[pl.ds(l, 1), pl.ds(0, 4), pl.ds(wid, 1)],
            sem,
        ).wait()

    start_gather(0, 0, sg0)

    def p2_body(k2, carry):
        l0 = 2 * k2
        l1 = l0 + 1
        start_gather(l1, 1, sg1)
        wait_gather(l0, 0, sg0)

        @pl.when(k2 > 0)
        def _():
            wait_store(l0 - 2, 0, ss0)

        rearrange(l0, 0)
        store_out(l0, 0, ss0)

        @pl.when(l0 + 2 < _LSEQ)
        def _():
            start_gather(l0 + 2, 0, sg0)

        wait_gather(l1, 1, sg1)

        @pl.when(k2 > 0)
        def _():
            wait_store(l1 - 2, 1, ss1)

        rearrange(l1, 1)
        store_out(l1, 1, ss1)
        return carry

    lax.fori_loop(0, _LSEQ // 2, p2_body, 0)

    wait_store(_LSEQ - 2, 0, ss0)
    wait_store(_LSEQ - 1, 1, ss1)


def kernel(x, table):
    xT = x.astype(jnp.int32).T
    tT = table.T
    tail = jnp.pad(tT[:, _UFULL * 128 :], ((0, 0), (0, 64)))
    scratch = _transpose_sc(tT, tail)
    out5 = _gather_sc(xT, scratch)
    out = out5.transpose(2, 4, 0, 1, 3).reshape(_BATCH, _LSEQ, _D)
    return out


# speedup vs baseline: 2.3833x; 2.1378x over previous
import functools
import math

import jax
import jax.numpy as jnp
from jax import lax
from jax.experimental import pallas as pl
from jax.experimental.pallas import tpu as pltpu
from jax.experimental.pallas import tpu_sc as plsc

_BATCH = 4096
_LSEQ = 50
_D = 32
_NC, _NS = 2, 16
_NW = _NC * _NS
_BPW = _BATCH // _NW
_UFULL = 7812
_PAIRS = _UFULL // (2 * _NW)
_SQRT_S = math.sqrt(32.0)
_L = 16
_SROWS = 250016

_MESH = plsc.VectorSubcoreMesh(core_axis_name="c", subcore_axis_name="s")
_PARAMS = pltpu.CompilerParams(needs_layout_passes=False)


def _worker_id():
    return lax.axis_index("s") * _NC + lax.axis_index("c")


@functools.partial(
    pl.kernel,
    mesh=_MESH,
    compiler_params=_PARAMS,
    out_type=jax.ShapeDtypeStruct((_SROWS, 128), jnp.float32),
    scratch_types=[
        pltpu.VMEM((2, _D, 128), jnp.float32),
        pltpu.VMEM((2, _D, 128), jnp.float32),
        pltpu.SemaphoreType.DMA,
        pltpu.SemaphoreType.DMA,
        pltpu.SemaphoreType.DMA,
        pltpu.SemaphoreType.DMA,
    ],
)
def _transpose_sc(tT_hbm, tail_hbm, scr_hbm, bin_v, bout_v, sf0, sf1, sw0, sw1):
    wid = _worker_id()
    lanes = lax.iota(jnp.int32, _L)

    def transpose_block(ib, ob, ncg):
        @plsc.parallel_loop(0, ncg, unroll=1)
        def _(cg):
            c16 = cg * _L + lanes
            drow = lax.shift_right_logical(c16, 2)
            k16 = jnp.bitwise_and(c16, 3)
            dcol = lax.shift_left(k16, 5)
            k4 = lax.shift_left(k16, 2)
            for d0 in range(0, _D, 8):
                vals = [
                    plsc.load_gather(
                        ib, [jnp.broadcast_to(jnp.int32(d), (_L,)), c16]
                    )
                    for d in range(d0, d0 + 8)
                ]
                for i, d in enumerate(range(d0, d0 + 8)):
                    scol = dcol + jnp.bitwise_and(k4 + d, 31)
                    plsc.store_scatter(ob, [drow, scol], vals[i] * _SQRT_S)

    def pair_body(k2, carry):
        u_a = wid + (2 * k2) * _NW
        u_b = u_a + _NW
        fa = pltpu.async_copy(
            tT_hbm.at[:, pl.ds(u_a * 128, 128)], bin_v.at[0], sf0
        )
        fb = pltpu.async_copy(
            tT_hbm.at[:, pl.ds(u_b * 128, 128)], bin_v.at[1], sf1
        )
        fa.wait()
        transpose_block(bin_v.at[0], bout_v.at[0], 8)
        wa = pltpu.async_copy(
            bout_v.at[0], scr_hbm.at[pl.ds(u_a * _D, _D)], sw0
        )
        fb.wait()
        transpose_block(bin_v.at[1], bout_v.at[1], 8)
        wb = pltpu.async_copy(
            bout_v.at[1], scr_hbm.at[pl.ds(u_b * _D, _D)], sw1
        )
        wa.wait()
        wb.wait()
        return carry

    lax.fori_loop(0, _PAIRS, pair_body, 0)

    @pl.when(wid < 4)
    def _():
        u = _UFULL - 4 + wid
        pltpu.async_copy(
            tT_hbm.at[:, pl.ds(u * 128, 128)], bin_v.at[0], sf0
        ).wait()
        transpose_block(bin_v.at[0], bout_v.at[0], 8)
        pltpu.async_copy(
            bout_v.at[0], scr_hbm.at[pl.ds(u * _D, _D)], sw0
        ).wait()

    @pl.when(wid == 4)
    def _():
        pltpu.async_copy(tail_hbm, bin_v.at[0], sf0).wait()
        transpose_block(bin_v.at[0], bout_v.at[0], 8)
        pltpu.async_copy(
            bout_v.at[0], scr_hbm.at[pl.ds(_UFULL * _D, _D)], sw0
        ).wait()


@functools.partial(
    pl.kernel,
    mesh=_MESH,
    compiler_params=_PARAMS,
    out_type=jax.ShapeDtypeStruct((_LSEQ, 4, _NW, 8, _BPW), jnp.float32),
    scratch_types=[
        pltpu.VMEM((_LSEQ, _BPW), jnp.int32),
        pltpu.VMEM((_LSEQ * _BPW,), jnp.int32),
        pltpu.VMEM((_LSEQ * _BPW,), jnp.int32),
        pltpu.VMEM((2, _BPW, 128), jnp.float32),
        pltpu.VMEM((2, 1, 4, 1, 8, _BPW), jnp.float32),
        pltpu.SemaphoreType.DMA,
        pltpu.SemaphoreType.DMA,
        pltpu.SemaphoreType.DMA,
        pltpu.SemaphoreType.DMA,
    ],
)
def _gather_sc(
    xT_hbm, scr_hbm, out_hbm,
    xsl_v, m_v, sub_v, wide_v, outc_v, sg0, sg1, ss0, ss1,
):
    wid = _worker_id()
    lanes = lax.iota(jnp.int32, _L)

    pltpu.sync_copy(xT_hbm.at[:, pl.ds(wid * _BPW, _BPW)], xsl_v)

    @plsc.parallel_loop(0, _LSEQ * _BPW // _L, unroll=1)
    def _(g):
        r = lax.shift_right_logical(g, 3)
        col = jnp.bitwise_and(g, 7) * _L
        v = xsl_v[r, pl.ds(col, _L)]
        m_v[pl.ds(g * _L, _L)] = lax.shift_right_logical(v, 2)
        sub_v[pl.ds(g * _L, _L)] = lax.shift_left(jnp.bitwise_and(v, 3), 5)

    def start_gather(l, buf, sem):
        return pltpu.async_copy(
            scr_hbm.at[m_v.at[pl.ds(l * _BPW, _BPW)]], wide_v.at[buf], sem
        )

    def wait_gather(l, buf, sem):
        pltpu.make_async_copy(
            scr_hbm.at[m_v.at[pl.ds(l * _BPW, _BPW)]], wide_v.at[buf], sem
        ).wait()

    def rearrange(l, buf):
        wbuf = wide_v.at[buf]
        obuf = outc_v.at[buf]

        @plsc.parallel_loop(0, _BPW // _L, unroll=1)
        def _(cg):
            rows16 = cg * _L + lanes
            sub16 = sub_v[pl.ds(l * _BPW + cg * _L, _L)]
            k4 = lax.shift_right_logical(sub16, 3)
            cslice = pl.ds(cg * _L, _L)
            for d0 in range(0, _D, 8):
                vals = [
                    plsc.load_gather(
                        wbuf, [rows16, sub16 + jnp.bitwise_and(k4 + d, 31)]
                    )
                    for d in range(d0, d0 + 8)
                ]
                for i, d in enumerate(range(d0, d0 + 8)):
                    obuf[0, d >> 3, 0, d & 7, cslice] = vals[i]

    def store_out(l, buf, sem):
        return pltpu.async_copy(
            outc_v.at[buf],
            out_hbm.at[pl.ds(l, 1), pl.ds(0, 4), pl.ds(wid, 1)],
            sem,
        )

    def wait_store(l, buf, sem):
        pltpu.make_async_copy(
            outc_v.at[buf],
            out_hbm.at[pl.ds(l, 1), pl.ds(0, 4), pl.ds(wid, 1)],
            sem,
        ).wait()

    start_gather(0, 0, sg0)

    def p2_body(k2, carry):
        l0 = 2 * k2
        l1 = l0 + 1
        start_gather(l1, 1, sg1)
        wait_gather(l0, 0, sg0)

        @pl.when(k2 > 0)
        def _():
            wait_store(l0 - 2, 0, ss0)

        rearrange(l0, 0)
        store_out(l0, 0, ss0)

        @pl.when(l0 + 2 < _LSEQ)
        def _():
            start_gather(l0 + 2, 0, sg0)

        wait_gather(l1, 1, sg1)

        @pl.when(k2 > 0)
        def _():
            wait_store(l1 - 2, 1, ss1)

        rearrange(l1, 1)
        store_out(l1, 1, ss1)
        return carry

    lax.fori_loop(0, _LSEQ // 2, p2_body, 0)

    wait_store(_LSEQ - 2, 0, ss0)
    wait_store(_LSEQ - 1, 1, ss1)


def kernel(x, table):
    xT = x.astype(jnp.int32).T
    tT = table.T
    tail = jnp.pad(tT[:, _UFULL * 128 :], ((0, 0), (0, 64)))
    scratch = _transpose_sc(tT, tail)
    out5 = _gather_sc(xT, scratch)
    out = out5.transpose(2, 4, 0, 1, 3).reshape(_BATCH, _LSEQ, _D)
    return out
